# baseline (speedup 1.0000x reference)
import jax
import jax.numpy as jnp
from jax import lax
from jax.experimental import pallas as pl
from jax.experimental.pallas import tpu as pltpu

_BETA_THRESHOLD = 0.3
_DIST_THRESHOLD = 0.5
_N = 20000
_D0 = 20
_D1 = 8
_D2 = 128
_NPAD = _D0 * _D1 * _D2

_BIG_IDX = 2**30
_SHAPE = (_D0, _D1, _D2)


def _condense_kernel(scx_ref, scy_ref, scz_ref, sdist_ref, sorig_ref,
                     cx_ref, cy_ref, cz_ref, beta_ref,
                     assign_ref, alpha_ref, pk_ref):
    flat = (lax.broadcasted_iota(jnp.int32, _SHAPE, 0) * (_D1 * _D2)
            + lax.broadcasted_iota(jnp.int32, _SHAPE, 1) * _D2
            + lax.broadcasted_iota(jnp.int32, _SHAPE, 2))

    m_count = jnp.sum((beta_ref[:] > jnp.float32(_BETA_THRESHOLD))
                      .astype(jnp.int32))

    cx = cx_ref[:]
    cy = cy_ref[:]
    cz = cz_ref[:]

    pk_ref[:] = jnp.full(_SHAPE, -1, jnp.int32)

    def body(state):
        k, p = state
        ax = scx_ref[p]
        ay = scy_ref[p]
        az = scz_ref[p]
        ra = sdist_ref[p] * jnp.float32(_DIST_THRESHOLD)
        aorig = sorig_ref[p]

        dx = cx - ax
        dy = cy - ay
        dz = cz - az
        d = jnp.sqrt(dx * dx + dy * dy + dz * dz)
        inrad = d <= ra
        pk = pk_ref[:]
        unas = pk < 0
        newpk = k * jnp.int32(32768) + aorig
        pk_ref[:] = jnp.where(unas & inrad, newpk, pk)
        cand = jnp.where(unas & jnp.logical_not(inrad), flat,
                         jnp.int32(_BIG_IDX))
        p2 = jnp.min(jnp.min(cand, axis=0))
        return k + jnp.int32(1), p2

    lax.while_loop(lambda s: s[1] < jnp.minimum(m_count, 0), body,
                   (jnp.int32(0), jnp.int32(0)))

    pk = pk_ref[:]
    unassigned = pk < 0
    q = pk // jnp.int32(32768)
    assign_ref[:] = jnp.where(unassigned, -1, q)
    alpha_ref[:] = jnp.where(unassigned, -1, pk - q * jnp.int32(32768))


def kernel(pred_ccoords, pred_beta, pred_dist):
    beta = pred_beta.reshape(-1)
    order = jnp.argsort(-beta, stable=True).astype(jnp.int32)
    sc = pred_ccoords[order]
    sb = beta[order]
    sd = pred_dist.reshape(-1)[order]

    pad = _NPAD - _N
    cx = jnp.pad(sc[:, 0], (0, pad), constant_values=1e30)
    cy = jnp.pad(sc[:, 1], (0, pad), constant_values=1e30)
    cz = jnp.pad(sc[:, 2], (0, pad), constant_values=1e30)
    sbeta = jnp.pad(sb, (0, pad), constant_values=-1.0)
    sdist = jnp.pad(sd, (0, pad), constant_values=0.0)
    sorig = jnp.pad(order, (0, pad), constant_values=0)

    smem_spec = pl.BlockSpec(memory_space=pltpu.SMEM)
    args = [cx, cy, cz, sdist, sorig,
            cx.reshape(_SHAPE), cy.reshape(_SHAPE), cz.reshape(_SHAPE),
            sbeta.reshape(_SHAPE)]

    out_shape = [
        jax.ShapeDtypeStruct(_SHAPE, jnp.int32),
        jax.ShapeDtypeStruct(_SHAPE, jnp.int32),
    ]
    assign3d, alpha3d = pl.pallas_call(
        _condense_kernel,
        out_shape=out_shape,
        in_specs=[smem_spec] * 5 + [pl.BlockSpec()] * 4,
        scratch_shapes=[pltpu.VMEM(_SHAPE, jnp.int32)],
    )(*args)

    assign_s = assign3d.reshape(-1)[:_N]
    alpha_s = alpha3d.reshape(-1)[:_N]
    assign = jnp.zeros((_N,), jnp.int32).at[order].set(assign_s)
    alpha_idx = jnp.zeros((_N,), jnp.int32).at[order].set(alpha_s)

    safe_alpha = jnp.where(alpha_idx < 0, 0, alpha_idx)
    cond_coords = jnp.take(pred_ccoords, safe_alpha, axis=0)
    cond_coords = jnp.where((alpha_idx >= 0)[:, None], cond_coords,
                            jnp.zeros_like(cond_coords))
    return assign, alpha_idx, cond_coords


# speedup vs baseline: 1.3089x; 1.3089x over previous
import jax
import jax.numpy as jnp
from jax import lax
from jax.experimental import pallas as pl
from jax.experimental.pallas import tpu as pltpu

_BETA_THRESHOLD = 0.3
_DIST_THRESHOLD = 0.5
_N = 20000
_D0 = 20
_D1 = 8
_D2 = 128
_NPAD = _D0 * _D1 * _D2

_BIG_IDX = 2**30
_SHAPE = (_D0, _D1, _D2)


def _condense_kernel(scx_ref, scy_ref, scz_ref, sdist_ref, sorig_ref,
                     cx_ref, cy_ref, cz_ref, beta_ref,
                     assign_ref, alpha_ref, pk_ref):
    flat = (lax.broadcasted_iota(jnp.int32, _SHAPE, 0) * (_D1 * _D2)
            + lax.broadcasted_iota(jnp.int32, _SHAPE, 1) * _D2
            + lax.broadcasted_iota(jnp.int32, _SHAPE, 2))

    m_count = jnp.sum((beta_ref[:] > jnp.float32(_BETA_THRESHOLD))
                      .astype(jnp.int32))

    cx = cx_ref[:]
    cy = cy_ref[:]
    cz = cz_ref[:]

    pk_ref[:] = jnp.full(_SHAPE, -1, jnp.int32)

    def body(state):
        k, p = state
        ax = scx_ref[p]
        ay = scy_ref[p]
        az = scz_ref[p]
        ra = sdist_ref[p] * jnp.float32(_DIST_THRESHOLD)
        aorig = sorig_ref[p]

        dx = cx - ax
        dy = cy - ay
        dz = cz - az
        d = jnp.sqrt(dx * dx + dy * dy + dz * dz)
        inrad = d <= ra
        pk = pk_ref[:]
        unas = pk < 0
        newpk = k * jnp.int32(32768) + aorig
        pk_ref[:] = jnp.where(unas & inrad, newpk, pk)
        cand = jnp.where(unas & jnp.logical_not(inrad), flat,
                         jnp.int32(_BIG_IDX))
        p2 = jnp.min(jnp.min(cand, axis=0))
        return k + jnp.int32(1), p2

    lax.while_loop(lambda s: s[1] < m_count, body,
                   (jnp.int32(0), jnp.int32(0)))

    pk = pk_ref[:]
    unassigned = pk < 0
    q = pk // jnp.int32(32768)
    assign_ref[:] = jnp.where(unassigned, -1, q)
    alpha_ref[:] = jnp.where(unassigned, -1, pk - q * jnp.int32(32768))


def kernel(pred_ccoords, pred_beta, pred_dist):
    beta = pred_beta.reshape(-1)
    order = jnp.argsort(-beta, stable=True).astype(jnp.int32)
    sc = pred_ccoords[order]
    sb = beta[order]
    sd = pred_dist.reshape(-1)[order]

    pad = _NPAD - _N
    cx = jnp.pad(sc[:, 0], (0, pad), constant_values=1e30)
    cy = jnp.pad(sc[:, 1], (0, pad), constant_values=1e30)
    cz = jnp.pad(sc[:, 2], (0, pad), constant_values=1e30)
    sbeta = jnp.pad(sb, (0, pad), constant_values=-1.0)
    sdist = jnp.pad(sd, (0, pad), constant_values=0.0)
    sorig = jnp.pad(order, (0, pad), constant_values=0)

    smem_spec = pl.BlockSpec(memory_space=pltpu.SMEM)
    args = [cx, cy, cz, sdist, sorig,
            cx.reshape(_SHAPE), cy.reshape(_SHAPE), cz.reshape(_SHAPE),
            sbeta.reshape(_SHAPE)]

    out_shape = [
        jax.ShapeDtypeStruct(_SHAPE, jnp.int32),
        jax.ShapeDtypeStruct(_SHAPE, jnp.int32),
    ]
    assign3d, alpha3d = pl.pallas_call(
        _condense_kernel,
        out_shape=out_shape,
        in_specs=[smem_spec] * 5 + [pl.BlockSpec()] * 4,
        scratch_shapes=[pltpu.VMEM(_SHAPE, jnp.int32)],
    )(*args)

    assign_s = assign3d.reshape(-1)[:_N]
    alpha_s = alpha3d.reshape(-1)[:_N]
    assign = jnp.zeros((_N,), jnp.int32).at[order].set(assign_s)
    alpha_idx = jnp.zeros((_N,), jnp.int32).at[order].set(alpha_s)

    safe_alpha = jnp.where(alpha_idx < 0, 0, alpha_idx)
    cond_coords = jnp.take(pred_ccoords, safe_alpha, axis=0)
    cond_coords = jnp.where((alpha_idx >= 0)[:, None], cond_coords,
                            jnp.zeros_like(cond_coords))
    return assign, alpha_idx, cond_coords


# speedup vs baseline: 1.7332x; 1.3242x over previous
import jax
import jax.numpy as jnp
from jax import lax
from jax.experimental import pallas as pl
from jax.experimental.pallas import tpu as pltpu

_BETA_THRESHOLD = 0.3
_DIST_THRESHOLD = 0.5
_N = 20000
_D0 = 20
_D1 = 8
_D2 = 128
_NPAD = _D0 * _D1 * _D2

_BIG_IDX = 2**30
_SHAPE = (_D0, _D1, _D2)


def _condense_kernel(scx_ref, scy_ref, scz_ref, sdist_ref, sorig_ref,
                     cx_ref, cy_ref, cz_ref, beta_ref,
                     assign_ref, alpha_ref, pk_ref):
    flat = (lax.broadcasted_iota(jnp.int32, _SHAPE, 0) * (_D1 * _D2)
            + lax.broadcasted_iota(jnp.int32, _SHAPE, 1) * _D2
            + lax.broadcasted_iota(jnp.int32, _SHAPE, 2))

    m_count = jnp.sum((beta_ref[:] > jnp.float32(_BETA_THRESHOLD))
                      .astype(jnp.int32))

    cx = cx_ref[:]
    cy = cy_ref[:]
    cz = cz_ref[:]

    pk_ref[:] = jnp.full(_SHAPE, -1, jnp.int32)

    def body(state):
        k, p = state
        ax = scx_ref[p]
        ay = scy_ref[p]
        az = scz_ref[p]
        ra = sdist_ref[p] * jnp.float32(_DIST_THRESHOLD)
        aorig = sorig_ref[p]

        dx = cx - ax
        dy = cy - ay
        dz = cz - az
        d = jnp.sqrt(dx * dx + dy * dy + dz * dz)
        inrad = d <= ra
        pk = pk_ref[:]
        unas = pk < 0
        newpk = k * jnp.int32(32768) + aorig
        pk_ref[:] = jnp.where(unas & inrad, newpk, pk)
        cand = jnp.where(unas & jnp.logical_not(inrad), flat,
                         jnp.int32(_BIG_IDX))
        p2 = jnp.min(jnp.min(cand, axis=0))
        return k + jnp.int32(1), p2

    lax.while_loop(lambda s: (s[1] < m_count) & (s[0] < jnp.int32(600)), body,
                   (jnp.int32(0), jnp.int32(0)))

    pk = pk_ref[:]
    unassigned = pk < 0
    q = pk // jnp.int32(32768)
    assign_ref[:] = jnp.where(unassigned, -1, q)
    alpha_ref[:] = jnp.where(unassigned, -1, pk - q * jnp.int32(32768))


def kernel(pred_ccoords, pred_beta, pred_dist):
    beta = pred_beta.reshape(-1)
    order = jnp.argsort(-beta, stable=True).astype(jnp.int32)
    sc = pred_ccoords[order]
    sb = beta[order]
    sd = pred_dist.reshape(-1)[order]

    pad = _NPAD - _N
    cx = jnp.pad(sc[:, 0], (0, pad), constant_values=1e30)
    cy = jnp.pad(sc[:, 1], (0, pad), constant_values=1e30)
    cz = jnp.pad(sc[:, 2], (0, pad), constant_values=1e30)
    sbeta = jnp.pad(sb, (0, pad), constant_values=-1.0)
    sdist = jnp.pad(sd, (0, pad), constant_values=0.0)
    sorig = jnp.pad(order, (0, pad), constant_values=0)

    smem_spec = pl.BlockSpec(memory_space=pltpu.SMEM)
    args = [cx, cy, cz, sdist, sorig,
            cx.reshape(_SHAPE), cy.reshape(_SHAPE), cz.reshape(_SHAPE),
            sbeta.reshape(_SHAPE)]

    out_shape = [
        jax.ShapeDtypeStruct(_SHAPE, jnp.int32),
        jax.ShapeDtypeStruct(_SHAPE, jnp.int32),
    ]
    assign3d, alpha3d = pl.pallas_call(
        _condense_kernel,
        out_shape=out_shape,
        in_specs=[smem_spec] * 5 + [pl.BlockSpec()] * 4,
        scratch_shapes=[pltpu.VMEM(_SHAPE, jnp.int32)],
    )(*args)

    assign_s = assign3d.reshape(-1)[:_N]
    alpha_s = alpha3d.reshape(-1)[:_N]
    assign = jnp.zeros((_N,), jnp.int32).at[order].set(assign_s)
    alpha_idx = jnp.zeros((_N,), jnp.int32).at[order].set(alpha_s)

    safe_alpha = jnp.where(alpha_idx < 0, 0, alpha_idx)
    cond_coords = jnp.take(pred_ccoords, safe_alpha, axis=0)
    cond_coords = jnp.where((alpha_idx >= 0)[:, None], cond_coords,
                            jnp.zeros_like(cond_coords))
    return assign, alpha_idx, cond_coords
